# baseline (speedup 1.0000x reference)
import jax
import jax.numpy as jnp
from jax import lax
from jax.experimental import pallas as pl

_TEMPERATURE = 11.3137
_NUM_SAMPLED = 1024
_EPS = 1e-06

_B = 4
_S = 4096
_SV = 4095
_NEARLY = _NUM_SAMPLED + 1
_ETILE = 128
_NET = 9


def _rank_mask_kernel(sig_ref, out_ref):
    et = pl.program_id(1)
    row = sig_ref[0, 0, :]
    e0 = et * _ETILE
    ev = sig_ref[0, 0, pl.ds(e0, _ETILE)]
    eidx = e0 + lax.broadcasted_iota(jnp.int32, (_ETILE, 1), 0)
    jidx = lax.broadcasted_iota(jnp.int32, (_ETILE, _SV), 1)
    allv = row[None, :]
    evc = ev[:, None]
    le = allv <= evc
    lt = allv < evc
    jlt = jidx < eidx
    pick = (le & jlt) | lt
    cnt = jnp.dot(
        jnp.where(pick, 1.0, 0.0).astype(jnp.bfloat16),
        jnp.ones((_SV,), jnp.bfloat16),
        preferred_element_type=jnp.float32,
    ).astype(jnp.int32)
    valid = (e0 + lax.iota(jnp.int32, _ETILE)) < _NEARLY
    pos = jnp.where(valid, cnt + 1, 0)
    jfull = lax.broadcasted_iota(jnp.int32, (_ETILE, _S), 1)
    hit = jnp.any(jfull == pos[:, None], axis=0)
    lane = lax.broadcasted_iota(jnp.int32, (_S,), 0)

    @pl.when(et == 0)
    def _():
        out_ref[0, 0, :] = (lane == 0) | hit

    @pl.when(et > 0)
    def _():
        out_ref[0, 0, :] = out_ref[0, 0, :] | hit


def kernel(q, k, v, token_mask):
    attn = jnp.matmul(q[..., :1, :], jnp.swapaxes(k, -2, -1)) / _TEMPERATURE
    attn = attn - jnp.max(attn, axis=-1, keepdims=True)
    batch_size, seq_length = token_mask.shape
    attn_mask = token_mask.reshape(batch_size, 1, 1, seq_length)
    attn = jnp.exp(attn) * attn_mask
    attn = (attn + _EPS / seq_length) / (jnp.sum(attn, axis=-1, keepdims=True) + _EPS)
    sig = jnp.sum(attn[:, :, 0], axis=1)
    sig = sig[:, 1:]
    sig = sig / jnp.sum(sig, axis=1, keepdims=True)

    mask = pl.pallas_call(
        _rank_mask_kernel,
        grid=(_B, _NET),
        in_specs=[pl.BlockSpec((1, 1, _SV), lambda b, e: (b, 0, 0))],
        out_specs=pl.BlockSpec((1, 1, _S), lambda b, e: (b, 0, 0)),
        out_shape=jax.ShapeDtypeStruct((_B, 1, _S), jnp.bool_),
    )(sig.reshape(_B, 1, _SV))
    return mask.reshape(_B, _S)
